# Initial kernel scaffold; baseline (speedup 1.0000x reference)
import jax
import jax.numpy as jnp
from jax.experimental import pallas as pl


def kernel(points, features, lorentz_vectors, mask, params):
    raise NotImplementedError("write your pallas kernel here")


# speedup vs baseline: 58.1559x; 58.1559x over previous
import functools

import jax
import jax.numpy as jnp
import numpy as np
from jax.experimental import pallas as pl

_B, _N, _K = 128, 128, 16
_DIMS = [(34, 64)] + [(64, 64)] * 3 + [(64, 128)] + [(128, 128)] * 3 + [(128, 256)] + [(256, 256)] * 3
_MLP = [(256, 128), (128, 64), (64, 5)]
_BN = float(1.0 / np.sqrt(1.0 + 1e-5))
_IN_NORM = float(_K) ** -0.5


def _body(pts_ref, ptsT_ref, mask_ref, feat_ref, *refs):
    out_ref = refs[-1]
    wrefs = refs[:-1]

    pts = pts_ref[0]
    ptsT = ptsT_ref[0]
    dx = pts[:, 0:1] - ptsT[0:1, :]
    dy = pts[:, 1:2] - ptsT[1:2, :]
    d2 = dx * dx + dy * dy

    jcol = jax.lax.broadcasted_iota(jnp.int32, (_N, _N), 1)
    inf = jnp.float32(np.inf)
    work = d2
    adj = jnp.zeros((_N, _N), dtype=jnp.float32)
    for t in range(_K + 1):
        m = jnp.min(work, axis=1, keepdims=True)
        is_min = work == m
        jsel = jnp.min(jnp.where(is_min, jcol, _N), axis=1, keepdims=True)
        onehot = jcol == jsel
        if t >= 1:
            adj = adj + onehot.astype(jnp.float32)
        work = jnp.where(onehot, inf, work)

    deg = jnp.sum(adj, axis=0, keepdims=True)
    out_norm = jax.lax.rsqrt(jnp.maximum(deg, 1.0))
    at = adj * (out_norm * _IN_NORM)

    mask = mask_ref[0]
    h = feat_ref[0] * mask * _BN
    idx = 0
    for din, dout in _DIMS:
        w = wrefs[idx][...]
        b = wrefs[idx + 1][...]
        idx += 2
        h = jax.nn.relu(h * _BN)
        agg = jax.lax.dot(at, h, preferred_element_type=jnp.float32)
        hw = jax.lax.dot(agg, w, preferred_element_type=jnp.float32) + b
        h = h + hw if din == dout else hw

    hg = jnp.sum(h, axis=0, keepdims=True) * (1.0 / _N)
    for li, (din, dout) in enumerate(_MLP):
        w = wrefs[idx][...]
        b = wrefs[idx + 1][...]
        idx += 2
        hg = jax.lax.dot(hg, w, preferred_element_type=jnp.float32) + b
        if li < len(_MLP) - 1:
            hg = jax.nn.relu(hg)
    out_ref[0] = hg


@jax.jit
def kernel(points, features, lorentz_vectors, mask, params):
    del lorentz_vectors
    ptsT = jnp.swapaxes(points, 1, 2)
    mask3 = mask[:, :, None]

    weights = []
    in_specs = [
        pl.BlockSpec((1, _N, 2), lambda b: (b, 0, 0)),
        pl.BlockSpec((1, 2, _N), lambda b: (b, 0, 0)),
        pl.BlockSpec((1, _N, 1), lambda b: (b, 0, 0)),
        pl.BlockSpec((1, _N, 34), lambda b: (b, 0, 0)),
    ]
    for i, (din, dout) in enumerate(_DIMS):
        weights.append(params['W%d' % i])
        weights.append(params['b%d' % i].reshape(1, dout))
        in_specs.append(pl.BlockSpec((din, dout), lambda b: (0, 0)))
        in_specs.append(pl.BlockSpec((1, dout), lambda b: (0, 0)))
    for i, (din, dout) in enumerate(_MLP):
        weights.append(params['Wm%d' % i])
        weights.append(params['bm%d' % i].reshape(1, dout))
        in_specs.append(pl.BlockSpec((din, dout), lambda b: (0, 0)))
        in_specs.append(pl.BlockSpec((1, dout), lambda b: (0, 0)))

    out = pl.pallas_call(
        _body,
        grid=(_B,),
        in_specs=in_specs,
        out_specs=pl.BlockSpec((1, 1, 5), lambda b: (b, 0, 0)),
        out_shape=jax.ShapeDtypeStruct((_B, 1, 5), jnp.float32),
    )(points, ptsT, mask3, features, *weights)
    return out.reshape(_B, 5)


# speedup vs baseline: 271.8799x; 4.6750x over previous
import functools

import jax
import jax.numpy as jnp
import numpy as np
from jax.experimental import pallas as pl

_B, _N, _K = 128, 128, 16
_BB = 8
_DIMS = [(34, 64)] + [(64, 64)] * 3 + [(64, 128)] + [(128, 128)] * 3 + [(128, 256)] + [(256, 256)] * 3
_MLP = [(256, 128), (128, 64), (64, 5)]
_BN = float(1.0 / np.sqrt(1.0 + 1e-5))
_IN_NORM = float(_K) ** -0.5


def _body(pts_ref, ptsT_ref, mask_ref, feat_ref, *refs):
    out_ref = refs[-1]
    wrefs = refs[:-1]

    pts = pts_ref[...]
    ptsT = ptsT_ref[...]
    dx = pts[:, :, 0:1] - ptsT[:, 0:1, :]
    dy = pts[:, :, 1:2] - ptsT[:, 1:2, :]
    d2 = dx * dx + dy * dy

    jcol = jax.lax.broadcasted_iota(jnp.int32, (_BB, _N, _N), 2)
    inf = jnp.float32(np.inf)
    work = d2
    adj = jnp.zeros((_BB, _N, _N), dtype=jnp.float32)
    for t in range(_K + 1):
        m = jnp.min(work, axis=2, keepdims=True)
        is_min = work == m
        jsel = jnp.min(jnp.where(is_min, jcol, _N), axis=2, keepdims=True)
        onehot = jcol == jsel
        if t >= 1:
            adj = adj + onehot.astype(jnp.float32)
        work = jnp.where(onehot, inf, work)

    deg = jnp.sum(adj, axis=1, keepdims=True)
    out_norm = jax.lax.rsqrt(jnp.maximum(deg, 1.0))
    at = adj * (out_norm * _IN_NORM)

    mask = mask_ref[...]
    h = feat_ref[...] * mask * _BN
    idx = 0
    for din, dout in _DIMS:
        w = wrefs[idx][...]
        b = wrefs[idx + 1][...]
        idx += 2
        h = jax.nn.relu(h * _BN)
        agg = jax.lax.dot_general(
            at, h, (((2,), (1,)), ((0,), (0,))),
            preferred_element_type=jnp.float32)
        hw = jax.lax.dot_general(
            agg, w, (((2,), (0,)), ((), ())),
            preferred_element_type=jnp.float32) + b
        h = h + hw if din == dout else hw

    hg = jnp.sum(h, axis=1) * (1.0 / _N)
    for li, (din, dout) in enumerate(_MLP):
        w = wrefs[idx][...]
        b = wrefs[idx + 1][...]
        idx += 2
        hg = jax.lax.dot(hg, w, preferred_element_type=jnp.float32) + b[0]
        if li < len(_MLP) - 1:
            hg = jax.nn.relu(hg)
    out_ref[...] = hg


@jax.jit
def kernel(points, features, lorentz_vectors, mask, params):
    del lorentz_vectors
    ptsT = jnp.swapaxes(points, 1, 2)
    mask3 = mask[:, :, None]

    weights = []
    in_specs = [
        pl.BlockSpec((_BB, _N, 2), lambda b: (b, 0, 0)),
        pl.BlockSpec((_BB, 2, _N), lambda b: (b, 0, 0)),
        pl.BlockSpec((_BB, _N, 1), lambda b: (b, 0, 0)),
        pl.BlockSpec((_BB, _N, 34), lambda b: (b, 0, 0)),
    ]
    for i, (din, dout) in enumerate(_DIMS):
        weights.append(params['W%d' % i])
        weights.append(params['b%d' % i].reshape(1, dout))
        in_specs.append(pl.BlockSpec((din, dout), lambda b: (0, 0)))
        in_specs.append(pl.BlockSpec((1, dout), lambda b: (0, 0)))
    for i, (din, dout) in enumerate(_MLP):
        weights.append(params['Wm%d' % i])
        weights.append(params['bm%d' % i].reshape(1, dout))
        in_specs.append(pl.BlockSpec((din, dout), lambda b: (0, 0)))
        in_specs.append(pl.BlockSpec((1, dout), lambda b: (0, 0)))

    out = pl.pallas_call(
        _body,
        grid=(_B // _BB,),
        in_specs=in_specs,
        out_specs=pl.BlockSpec((_BB, 5), lambda b: (b, 0)),
        out_shape=jax.ShapeDtypeStruct((_B, 5), jnp.float32),
    )(points, ptsT, mask3, features, *weights)
    return out


# speedup vs baseline: 456.0725x; 1.6775x over previous
import functools

import jax
import jax.numpy as jnp
import numpy as np
from jax.experimental import pallas as pl
from jax.experimental.pallas import tpu as pltpu

_B, _N, _K = 128, 128, 16
_BB = 16
_NSTEP = _B // _BB
_DIMS = [(34, 64)] + [(64, 64)] * 3 + [(64, 128)] + [(128, 128)] * 3 + [(128, 256)] + [(256, 256)] * 3
_MLP = [(256, 128), (128, 64), (64, 5)]
_BN = float(1.0 / np.sqrt(1.0 + 1e-5))
_IN_NORM = float(_K) ** -0.5


def _body(pts_ref, ptsT_ref, mask_ref, feat_ref, *refs):
    out_ref = refs[-2]
    at_ref = refs[-1]
    wrefs = refs[:-2]

    at = at_ref[...]
    mask = mask_ref[...]
    h = feat_ref[...] * mask * _BN
    idx = 0
    for din, dout in _DIMS:
        w = wrefs[idx][...]
        b = wrefs[idx + 1][...]
        idx += 2
        h = jax.nn.relu(h * _BN)
        agg = jax.lax.dot_general(
            at, h, (((2,), (1,)), ((0,), (0,))),
            preferred_element_type=jnp.float32)
        hw = jax.lax.dot_general(
            agg, w, (((2,), (0,)), ((), ())),
            preferred_element_type=jnp.float32) + b
        h = h + hw if din == dout else hw

    hg = jnp.sum(h, axis=1) * (1.0 / _N)
    for li, (din, dout) in enumerate(_MLP):
        w = wrefs[idx][...]
        b = wrefs[idx + 1][...]
        idx += 2
        hg = jax.lax.dot(hg, w, preferred_element_type=jnp.float32) + b[0]
        if li < len(_MLP) - 1:
            hg = jax.nn.relu(hg)
    out_ref[...] = hg

    pts = pts_ref[...]
    ptsT = ptsT_ref[...]
    dx = pts[:, :, 0:1] - ptsT[:, 0:1, :]
    dy = pts[:, :, 1:2] - ptsT[:, 1:2, :]
    work = dx * dx + dy * dy

    jcol = jax.lax.broadcasted_iota(jnp.int32, (_BB, _N, _N), 2).astype(jnp.float32)
    inf = jnp.float32(np.inf)
    rank0 = None
    for t in range(_K + 1):
        m = jnp.min(work, axis=2, keepdims=True)
        is_min = work == m
        jsel = jnp.min(jnp.where(is_min, jcol, jnp.float32(_N)), axis=2,
                       keepdims=True)
        onehot = jcol == jsel
        if t == 0:
            rank0 = onehot
        work = jnp.where(onehot, inf, work)
    adj = jnp.where(work == inf, 1.0, 0.0) - rank0.astype(jnp.float32)

    deg = jnp.sum(adj, axis=1, keepdims=True)
    out_norm = jax.lax.rsqrt(jnp.maximum(deg, 1.0))
    at_ref[...] = adj * (out_norm * _IN_NORM)


@jax.jit
def kernel(points, features, lorentz_vectors, mask, params):
    del lorentz_vectors
    ptsT = jnp.swapaxes(points, 1, 2)
    mask3 = mask[:, :, None]

    last = _NSTEP - 1

    def sel_map(g):
        return (jnp.minimum(g, last), 0, 0)

    def lay_map3(g):
        return (jnp.maximum(g - 1, 0), 0, 0)

    def lay_map2(g):
        return (jnp.maximum(g - 1, 0), 0)

    weights = []
    in_specs = [
        pl.BlockSpec((_BB, _N, 2), sel_map),
        pl.BlockSpec((_BB, 2, _N), sel_map),
        pl.BlockSpec((_BB, _N, 1), lay_map3),
        pl.BlockSpec((_BB, _N, 34), lay_map3),
    ]
    for i, (din, dout) in enumerate(_DIMS):
        weights.append(params['W%d' % i])
        weights.append(params['b%d' % i].reshape(1, dout))
        in_specs.append(pl.BlockSpec((din, dout), lambda g: (0, 0)))
        in_specs.append(pl.BlockSpec((1, dout), lambda g: (0, 0)))
    for i, (din, dout) in enumerate(_MLP):
        weights.append(params['Wm%d' % i])
        weights.append(params['bm%d' % i].reshape(1, dout))
        in_specs.append(pl.BlockSpec((din, dout), lambda g: (0, 0)))
        in_specs.append(pl.BlockSpec((1, dout), lambda g: (0, 0)))

    out = pl.pallas_call(
        _body,
        grid=(_NSTEP + 1,),
        in_specs=in_specs,
        out_specs=pl.BlockSpec((_BB, 5), lay_map2),
        out_shape=jax.ShapeDtypeStruct((_B, 5), jnp.float32),
        scratch_shapes=[pltpu.VMEM((_BB, _N, _N), jnp.float32)],
    )(points, ptsT, mask3, features, *weights)
    return out
